# Initial kernel scaffold; baseline (speedup 1.0000x reference)
import jax
import jax.numpy as jnp
from jax.experimental import pallas as pl


def kernel(x, edge_index, W1, b1, W2, b2, W3, b3, W4, b4, W5, b5):
    raise NotImplementedError("write your pallas kernel here")


# speedup vs baseline: 5.9138x; 5.9138x over previous
import functools

import jax
import jax.numpy as jnp
from jax import lax
from jax.experimental import pallas as pl
from jax.experimental.pallas import tpu as pltpu
from jax.experimental.pallas import tpu_sc as plsc

N = 100000
E = 1600000
LANE = 16
N_SC = 2
N_TILES = 16
BLK = 512
ROW = 128
NDMA = BLK // ROW
EPT = -(-E // (N_TILES * BLK)) * BLK
E_PAD = EPT * N_TILES
NPT = 6256
NPT_LAST = N - 15 * NPT
RND = 1024
ACC_R = N + 16

_MESH = plsc.VectorSubcoreMesh(core_axis_name="c", subcore_axis_name="s")


def _build_widx(widx, base_node, nc, ch, iota_nc):
    for k in range(RND // ROW):
        for q in range(ROW // LANE):
            off = (base_node + k * ROW + q * LANE) * nc + ch
            widx[k, pl.ds(q * LANE, LANE)] = iota_nc + off


def _round_base(s, m):
    last = jnp.where(s == N_TILES - 1, NPT_LAST - RND, NPT - RND)
    off = jnp.where(m == 6, last, m * RND)
    return pl.multiple_of(s * NPT + off, 16)


def _agg_body(nc, gflat, src2, dst2, oflat, acc, sidx, didx, gidx, rows,
              widx, bounce, sem):
    c = lax.axis_index("c")
    s = lax.axis_index("s")
    nch = nc // N_SC
    iota_nc = lax.iota(jnp.int32, LANE) * nc

    def chunk(j, carry):
        ch = j * N_SC + c
        def init_round(m, carry2):
            b = _round_base(s, m)
            _build_widx(widx, b, nc, ch, iota_nc)
            cps = [
                pltpu.async_copy(gflat.at[widx.at[k]],
                                 bounce.at[pl.ds(k * ROW, ROW)], sem)
                for k in range(RND // ROW)
            ]
            for cp in cps:
                cp.wait()
            pltpu.sync_copy(bounce, acc.at[pl.ds(b, RND)])
            return carry2

        lax.fori_loop(0, 7, init_round, 0)
        plsc.subcore_barrier()

        def blk(ib, carry2):
            row = s * (EPT // ROW) + ib * NDMA
            pltpu.sync_copy(src2.at[pl.ds(row, NDMA)], sidx)
            pltpu.sync_copy(dst2.at[pl.ds(row, NDMA)], didx)
            for k in range(NDMA):
                for q in range(ROW // LANE):
                    sl = pl.ds(q * LANE, LANE)
                    gidx[k, sl] = sidx[k, sl] * nc + ch
            cps = [
                pltpu.async_copy(gflat.at[gidx.at[k]],
                                 rows.at[pl.ds(k * ROW, ROW)], sem)
                for k in range(NDMA)
            ]
            for cp in cps:
                cp.wait()
            for k in range(NDMA):
                pltpu.sync_copy(rows.at[pl.ds(k * ROW, ROW)],
                                acc.at[didx.at[k]], add=True)
            return carry2

        lax.fori_loop(0, EPT // BLK, blk, 0)
        plsc.subcore_barrier()

        def wout_round(m, carry2):
            b = _round_base(s, m)
            _build_widx(widx, b, nc, ch, iota_nc)
            pltpu.sync_copy(acc.at[pl.ds(b, RND)], bounce)
            cps = [
                pltpu.async_copy(bounce.at[pl.ds(k * ROW, ROW)],
                                 oflat.at[widx.at[k]], sem)
                for k in range(RND // ROW)
            ]
            for cp in cps:
                cp.wait()
            return carry2

        lax.fori_loop(0, 7, wout_round, 0)
        plsc.subcore_barrier()
        return carry

    lax.fori_loop(0, nch, chunk, 0)


@functools.cache
def _make_agg(nc):
    return pl.kernel(
        functools.partial(_agg_body, nc),
        out_type=jax.ShapeDtypeStruct((N * nc, LANE), jnp.float32),
        mesh=_MESH,
        scratch_types=[
            pltpu.VMEM_SHARED((ACC_R, LANE), jnp.float32),
            pltpu.VMEM((NDMA, ROW), jnp.int32),
            pltpu.VMEM((NDMA, ROW), jnp.int32),
            pltpu.VMEM((NDMA, ROW), jnp.int32),
            pltpu.VMEM((BLK, LANE), jnp.float32),
            pltpu.VMEM((RND // ROW, ROW), jnp.int32),
            pltpu.VMEM((RND, LANE), jnp.float32),
            pltpu.SemaphoreType.DMA,
        ],
        compiler_params=pltpu.CompilerParams(use_tc_tiling_on_sc=False),
    )


def _deg_body(dst2, out2, acc, didx, buf):
    c = lax.axis_index("c")
    s = lax.axis_index("s")
    r0 = s * NPT
    ones = jnp.full((LANE,), 1.0, jnp.float32)
    for k in range(ROW):
        buf[k, :] = ones

    @pl.when(c == 0)
    def _init():
        for m in range(48):
            pltpu.sync_copy(buf, acc.at[pl.ds(r0 + m * ROW, ROW)])
        tail = r0 + pl.multiple_of(
            jnp.where(s == N_TILES - 1, NPT_LAST - ROW, NPT - ROW), 16)
        pltpu.sync_copy(buf, acc.at[pl.ds(tail, ROW)])
    plsc.subcore_barrier()

    @pl.when(c == 0)
    def _scatter():
        def blk(ib, carry):
            row = s * (EPT // ROW) + ib * NDMA
            pltpu.sync_copy(dst2.at[pl.ds(row, NDMA)], didx)
            for k in range(NDMA):
                pltpu.sync_copy(buf, acc.at[didx.at[k]], add=True)
            return carry
        lax.fori_loop(0, EPT // BLK, blk, 0)
    plsc.subcore_barrier()

    @pl.when(c == 0)
    def _wout():
        @pl.when(s < N_TILES - 1)
        def _a():
            pltpu.sync_copy(acc.at[pl.ds(r0, NPT)], out2.at[pl.ds(r0, NPT)])
        @pl.when(s == N_TILES - 1)
        def _b():
            pltpu.sync_copy(acc.at[pl.ds(r0, NPT_LAST)],
                            out2.at[pl.ds(r0, NPT_LAST)])


_deg_call = pl.kernel(
    _deg_body,
    out_type=jax.ShapeDtypeStruct((N, LANE), jnp.float32),
    mesh=_MESH,
    scratch_types=[
        pltpu.VMEM_SHARED((ACC_R, LANE), jnp.float32),
        pltpu.VMEM((NDMA, ROW), jnp.int32),
        pltpu.VMEM((ROW, LANE), jnp.float32),
    ],
    compiler_params=pltpu.CompilerParams(use_tc_tiling_on_sc=False),
)


BN = 2000
GRID = N // BN
_PREC = lax.Precision.HIGHEST


def _dinv(deg_ref):
    return lax.rsqrt(deg_ref[:, 0:1])


def _g0_body(x_ref, deg_ref, o_ref):
    o_ref[...] = x_ref[...] * _dinv(deg_ref)


def _post_body(A_ref, deg_ref, W_ref, b_ref, o_ref):
    dinv = _dinv(deg_ref)
    u = A_ref[...] * dinv
    y = jnp.dot(u, W_ref[...], preferred_element_type=jnp.float32,
                precision=_PREC) + b_ref[0:1, :]
    o_ref[...] = jnp.maximum(y, 0.0) * dinv


def _post4_body(A_ref, deg_ref, W4_ref, b4_ref, W5_ref, o_ref):
    dinv = _dinv(deg_ref)
    u = A_ref[...] * dinv
    y = jnp.dot(u, W4_ref[...], preferred_element_type=jnp.float32,
                precision=_PREC) + b4_ref[0:1, :]
    g4 = jnp.maximum(y, 0.0) * dinv
    o_ref[...] = jnp.dot(g4, W5_ref[...], preferred_element_type=jnp.float32,
                         precision=_PREC)


def _final_body(A_ref, deg_ref, b_ref, o_ref):
    o_ref[...] = A_ref[...] * _dinv(deg_ref) + b_ref[0:1, :]


def _row_spec(d):
    return pl.BlockSpec((BN, d), lambda i: (i, 0))


def _full_spec(r, c):
    return pl.BlockSpec((r, c), lambda i: (0, 0))


def _b2d(b):
    return jnp.broadcast_to(b.reshape(1, -1), (8, b.shape[0]))


def _tc_g0(x, deg16):
    return pl.pallas_call(
        _g0_body, grid=(GRID,),
        in_specs=[_row_spec(x.shape[1]), _row_spec(LANE)],
        out_specs=_row_spec(x.shape[1]),
        out_shape=jax.ShapeDtypeStruct((N, x.shape[1]), jnp.float32),
    )(x, deg16)


def _tc_post(A, deg16, W, b):
    din, dout = W.shape
    return pl.pallas_call(
        _post_body, grid=(GRID,),
        in_specs=[_row_spec(din), _row_spec(LANE),
                  _full_spec(din, dout), _full_spec(8, dout)],
        out_specs=_row_spec(dout),
        out_shape=jax.ShapeDtypeStruct((N, dout), jnp.float32),
    )(A, deg16, W, _b2d(b))


def _tc_post4(A, deg16, W4, b4, W5):
    return pl.pallas_call(
        _post4_body, grid=(GRID,),
        in_specs=[_row_spec(256), _row_spec(LANE), _full_spec(256, 512),
                  _full_spec(8, 512), _full_spec(512, 128)],
        out_specs=_row_spec(128),
        out_shape=jax.ShapeDtypeStruct((N, 128), jnp.float32),
    )(A, deg16, W4, _b2d(b4), W5)


def _tc_final(A, deg16, b):
    return pl.pallas_call(
        _final_body, grid=(GRID,),
        in_specs=[_row_spec(128), _row_spec(LANE), _full_spec(8, 128)],
        out_specs=_row_spec(128),
        out_shape=jax.ShapeDtypeStruct((N, 128), jnp.float32),
    )(A, deg16, _b2d(b))


def kernel(x, edge_index, W1, b1, W2, b2, W3, b3, W4, b4, W5, b5):
    src = edge_index[0].astype(jnp.int32)
    dst = edge_index[1].astype(jnp.int32)
    pad = E_PAD - E
    src2 = jnp.concatenate([src, jnp.zeros((pad,), jnp.int32)]
                           ).reshape(E_PAD // ROW, ROW)
    dst2 = jnp.concatenate([dst, jnp.full((pad,), N, jnp.int32)]
                           ).reshape(E_PAD // ROW, ROW)

    def agg(g, nc):
        gflat = g.reshape(N * nc, LANE)
        return _make_agg(nc)(gflat, src2, dst2).reshape(N, nc * LANE)

    deg16 = _deg_call(dst2)
    g = _tc_g0(x, deg16)
    g = _tc_post(agg(g, 2), deg16, W1, b1)
    g = _tc_post(agg(g, 4), deg16, W2, b2)
    g = _tc_post(agg(g, 8), deg16, W3, b3)
    z = _tc_post4(agg(g, 16), deg16, W4, b4, W5)
    return _tc_final(agg(z, 8), deg16, b5)


# speedup vs baseline: 9.6743x; 1.6359x over previous
import functools

import jax
import jax.numpy as jnp
from jax import lax
from jax.experimental import pallas as pl
from jax.experimental.pallas import tpu as pltpu
from jax.experimental.pallas import tpu_sc as plsc

N = 100000
E = 1600000
LANE = 16
N_SC = 2
N_TILES = 16
BLK = 512
ROW = 128
NDMA = BLK // ROW
EPT = -(-E // (N_TILES * BLK)) * BLK
E_PAD = EPT * N_TILES
NPT = 6256
NPT_LAST = N - 15 * NPT
RND = 512
NRND = 13
ACC_R = N + 16
NBLK = EPT // BLK
IDXROWS = E_PAD // ROW

_MESH = plsc.VectorSubcoreMesh(core_axis_name="c", subcore_axis_name="s")


def _build_widx(widx, base_node, nc, ch, iota_nc):
    for k in range(RND // ROW):
        for q in range(ROW // LANE):
            off = (base_node + k * ROW + q * LANE) * nc + ch
            widx[k, pl.ds(q * LANE, LANE)] = iota_nc + off


def _round_base(s, m):
    last = jnp.where(s == N_TILES - 1, NPT_LAST - RND, NPT - RND)
    off = jnp.where(m == NRND - 1, last, m * RND)
    return pl.multiple_of(s * NPT + off, 16)


def _agg_body(nc, gflat, src2, dst2, oflat, acc,
              sidxA, sidxB, gidxA, gidxB, didx0, didx1, didx2, didx3,
              rowsA, rowsB, widx, bounce, gsem, ssem, isem):
    c = lax.axis_index("c")
    s = lax.axis_index("s")
    nch = nc // N_SC
    iota_nc = lax.iota(jnp.int32, LANE) * nc
    sidx2 = (sidxA, sidxB)
    gidx2 = (gidxA, gidxB)
    didx4 = (didx0, didx1, didx2, didx3)
    rows2 = (rowsA, rowsB)

    def idx_row(jb):
        return jnp.minimum(s * (EPT // ROW) + jb * NDMA, IDXROWS - NDMA)

    def fire_idx(jb, sx, dx):
        r = idx_row(jb)
        pltpu.async_copy(src2.at[pl.ds(r, NDMA)], sx, isem)
        pltpu.async_copy(dst2.at[pl.ds(r, NDMA)], dx, isem)

    def wait_idx(sx, dx):
        pltpu.make_async_copy(src2.at[pl.ds(0, NDMA)], sx, isem).wait()
        pltpu.make_async_copy(dst2.at[pl.ds(0, NDMA)], dx, isem).wait()

    def compute_gidx(gx, sx, ch):
        for k in range(NDMA):
            for q in range(ROW // LANE):
                sl = pl.ds(q * LANE, LANE)
                gx[k, sl] = sx[k, sl] * nc + ch

    def fire_gathers(gx, rx):
        for k in range(NDMA):
            pltpu.async_copy(gflat.at[gx.at[k]],
                             rx.at[pl.ds(k * ROW, ROW)], gsem)

    def wait_gathers(gx, rx):
        for k in range(NDMA):
            pltpu.make_async_copy(gflat.at[gx.at[k]],
                                  rx.at[pl.ds(k * ROW, ROW)], gsem).wait()

    def fire_scatter(rx, dx):
        for k in range(NDMA):
            pltpu.async_copy(rx.at[pl.ds(k * ROW, ROW)],
                             acc.at[dx.at[k]], ssem, add=True)

    def wait_scatter(rx, dx):
        for k in range(NDMA):
            pltpu.make_async_copy(rx.at[pl.ds(k * ROW, ROW)],
                                  acc.at[dx.at[k]], ssem).wait()

    def chunk(j, carry):
        ch = j * N_SC + c
        def init_round(m, carry2):
            b = _round_base(s, m)
            _build_widx(widx, b, nc, ch, iota_nc)
            cps = [
                pltpu.async_copy(gflat.at[widx.at[k]],
                                 bounce.at[pl.ds(k * ROW, ROW)], gsem)
                for k in range(RND // ROW)
            ]
            for cp in cps:
                cp.wait()
            pltpu.sync_copy(bounce, acc.at[pl.ds(b, RND)])
            return carry2

        lax.fori_loop(0, NRND, init_round, 0)
        plsc.subcore_barrier()

        pltpu.sync_copy(src2.at[pl.ds(idx_row(0), NDMA)], sidxA)
        pltpu.sync_copy(dst2.at[pl.ds(idx_row(0), NDMA)], didx0)
        compute_gidx(gidxA, sidxA, ch)
        fire_gathers(gidxA, rowsA)
        fire_idx(1, sidxB, didx1)
        fire_idx(2, sidxA, didx2)

        def quad(i, carry2):
            jb = i * 4
            for t in range(4):
                X, Y = rows2[t % 2], rows2[(t + 1) % 2]
                gX, gY = gidx2[t % 2], gidx2[(t + 1) % 2]
                sY = sidx2[(t + 1) % 2]
                wait_gathers(gX, X)
                fire_scatter(X, didx4[t % 4])
                wait_idx(sY, didx4[(t + 1) % 4])
                compute_gidx(gY, sY, ch)
                if t == 0:
                    @pl.when(i > 0)
                    def _():
                        wait_scatter(Y, didx4[3])
                else:
                    wait_scatter(Y, didx4[(t + 3) % 4])
                fire_gathers(gY, Y)
                fire_idx(jb + t + 3,
                         sidx2[(t + 3) % 2], didx4[(t + 3) % 4])
            return carry2

        lax.fori_loop(0, NBLK // 4, quad, 0)
        wait_gathers(gidx2[NBLK % 2], rows2[NBLK % 2])
        wait_scatter(rows2[(NBLK - 1) % 2], didx4[(NBLK - 1) % 4])
        wait_idx(sidx2[(NBLK + 1) % 2], didx4[(NBLK + 1) % 4])
        wait_idx(sidx2[NBLK % 2], didx4[(NBLK + 2) % 4])
        plsc.subcore_barrier()

        def wout_round(m, carry2):
            b = _round_base(s, m)
            _build_widx(widx, b, nc, ch, iota_nc)
            pltpu.sync_copy(acc.at[pl.ds(b, RND)], bounce)
            cps = [
                pltpu.async_copy(bounce.at[pl.ds(k * ROW, ROW)],
                                 oflat.at[widx.at[k]], gsem)
                for k in range(RND // ROW)
            ]
            for cp in cps:
                cp.wait()
            return carry2

        lax.fori_loop(0, NRND, wout_round, 0)
        plsc.subcore_barrier()
        return carry

    lax.fori_loop(0, nch, chunk, 0)


@functools.cache
def _make_agg(nc):
    idx_t = pltpu.VMEM((NDMA, ROW), jnp.int32)
    rows_t = pltpu.VMEM((BLK, LANE), jnp.float32)
    return pl.kernel(
        functools.partial(_agg_body, nc),
        out_type=jax.ShapeDtypeStruct((N * nc, LANE), jnp.float32),
        mesh=_MESH,
        scratch_types=[
            pltpu.VMEM_SHARED((ACC_R, LANE), jnp.float32),
            idx_t, idx_t, idx_t, idx_t,
            idx_t, idx_t, idx_t, idx_t,
            rows_t, rows_t,
            pltpu.VMEM((RND // ROW, ROW), jnp.int32),
            pltpu.VMEM((RND, LANE), jnp.float32),
            pltpu.SemaphoreType.DMA,
            pltpu.SemaphoreType.DMA,
            pltpu.SemaphoreType.DMA,
        ],
        compiler_params=pltpu.CompilerParams(use_tc_tiling_on_sc=False),
    )


def _deg_body(dst2, out2, acc, didx, buf):
    c = lax.axis_index("c")
    s = lax.axis_index("s")
    r0 = s * NPT
    ones = jnp.full((LANE,), 1.0, jnp.float32)
    for k in range(ROW):
        buf[k, :] = ones

    @pl.when(c == 0)
    def _init():
        for m in range(48):
            pltpu.sync_copy(buf, acc.at[pl.ds(r0 + m * ROW, ROW)])
        tail = r0 + pl.multiple_of(
            jnp.where(s == N_TILES - 1, NPT_LAST - ROW, NPT - ROW), 16)
        pltpu.sync_copy(buf, acc.at[pl.ds(tail, ROW)])
    plsc.subcore_barrier()

    @pl.when(c == 0)
    def _scatter():
        def blk(ib, carry):
            row = s * (EPT // ROW) + ib * NDMA
            pltpu.sync_copy(dst2.at[pl.ds(row, NDMA)], didx)
            for k in range(NDMA):
                pltpu.sync_copy(buf, acc.at[didx.at[k]], add=True)
            return carry
        lax.fori_loop(0, EPT // BLK, blk, 0)
    plsc.subcore_barrier()

    @pl.when(c == 0)
    def _wout():
        @pl.when(s < N_TILES - 1)
        def _a():
            pltpu.sync_copy(acc.at[pl.ds(r0, NPT)], out2.at[pl.ds(r0, NPT)])
        @pl.when(s == N_TILES - 1)
        def _b():
            pltpu.sync_copy(acc.at[pl.ds(r0, NPT_LAST)],
                            out2.at[pl.ds(r0, NPT_LAST)])


_deg_call = pl.kernel(
    _deg_body,
    out_type=jax.ShapeDtypeStruct((N, LANE), jnp.float32),
    mesh=_MESH,
    scratch_types=[
        pltpu.VMEM_SHARED((ACC_R, LANE), jnp.float32),
        pltpu.VMEM((NDMA, ROW), jnp.int32),
        pltpu.VMEM((ROW, LANE), jnp.float32),
    ],
    compiler_params=pltpu.CompilerParams(use_tc_tiling_on_sc=False),
)


BN = 2000
GRID = N // BN
_PREC = lax.Precision.HIGHEST


def _dinv(deg_ref):
    return lax.rsqrt(deg_ref[:, 0:1])


def _g0_body(x_ref, deg_ref, o_ref):
    o_ref[...] = x_ref[...] * _dinv(deg_ref)


def _post_body(A_ref, deg_ref, W_ref, b_ref, o_ref):
    dinv = _dinv(deg_ref)
    u = A_ref[...] * dinv
    y = jnp.dot(u, W_ref[...], preferred_element_type=jnp.float32,
                precision=_PREC) + b_ref[0:1, :]
    o_ref[...] = jnp.maximum(y, 0.0) * dinv


def _post4_body(A_ref, deg_ref, W4_ref, b4_ref, W5_ref, o_ref):
    dinv = _dinv(deg_ref)
    u = A_ref[...] * dinv
    y = jnp.dot(u, W4_ref[...], preferred_element_type=jnp.float32,
                precision=_PREC) + b4_ref[0:1, :]
    g4 = jnp.maximum(y, 0.0) * dinv
    o_ref[...] = jnp.dot(g4, W5_ref[...], preferred_element_type=jnp.float32,
                         precision=_PREC)


def _final_body(A_ref, deg_ref, b_ref, o_ref):
    o_ref[...] = A_ref[...] * _dinv(deg_ref) + b_ref[0:1, :]


def _row_spec(d):
    return pl.BlockSpec((BN, d), lambda i: (i, 0))


def _full_spec(r, c):
    return pl.BlockSpec((r, c), lambda i: (0, 0))


def _b2d(b):
    return jnp.broadcast_to(b.reshape(1, -1), (8, b.shape[0]))


def _tc_g0(x, deg16):
    return pl.pallas_call(
        _g0_body, grid=(GRID,),
        in_specs=[_row_spec(x.shape[1]), _row_spec(LANE)],
        out_specs=_row_spec(x.shape[1]),
        out_shape=jax.ShapeDtypeStruct((N, x.shape[1]), jnp.float32),
    )(x, deg16)


def _tc_post(A, deg16, W, b):
    din, dout = W.shape
    return pl.pallas_call(
        _post_body, grid=(GRID,),
        in_specs=[_row_spec(din), _row_spec(LANE),
                  _full_spec(din, dout), _full_spec(8, dout)],
        out_specs=_row_spec(dout),
        out_shape=jax.ShapeDtypeStruct((N, dout), jnp.float32),
    )(A, deg16, W, _b2d(b))


def _tc_post4(A, deg16, W4, b4, W5):
    return pl.pallas_call(
        _post4_body, grid=(GRID,),
        in_specs=[_row_spec(256), _row_spec(LANE), _full_spec(256, 512),
                  _full_spec(8, 512), _full_spec(512, 128)],
        out_specs=_row_spec(128),
        out_shape=jax.ShapeDtypeStruct((N, 128), jnp.float32),
    )(A, deg16, W4, _b2d(b4), W5)


def _tc_final(A, deg16, b):
    return pl.pallas_call(
        _final_body, grid=(GRID,),
        in_specs=[_row_spec(128), _row_spec(LANE), _full_spec(8, 128)],
        out_specs=_row_spec(128),
        out_shape=jax.ShapeDtypeStruct((N, 128), jnp.float32),
    )(A, deg16, _b2d(b))


def kernel(x, edge_index, W1, b1, W2, b2, W3, b3, W4, b4, W5, b5):
    src = edge_index[0].astype(jnp.int32)
    dst = edge_index[1].astype(jnp.int32)
    pad = E_PAD - E
    src2 = jnp.concatenate([src, jnp.zeros((pad,), jnp.int32)]
                           ).reshape(E_PAD // ROW, ROW)
    dst2 = jnp.concatenate([dst, jnp.full((pad,), N, jnp.int32)]
                           ).reshape(E_PAD // ROW, ROW)

    def agg(g, nc):
        gflat = g.reshape(N * nc, LANE)
        return _make_agg(nc)(gflat, src2, dst2).reshape(N, nc * LANE)

    deg16 = _deg_call(dst2)
    g = _tc_g0(x, deg16)
    g = _tc_post(agg(g, 2), deg16, W1, b1)
    g = _tc_post(agg(g, 4), deg16, W2, b2)
    g = _tc_post(agg(g, 8), deg16, W3, b3)
    z = _tc_post4(agg(g, 16), deg16, W4, b4, W5)
    return _tc_final(agg(z, 8), deg16, b5)


# speedup vs baseline: 11.2702x; 1.1650x over previous
import functools

import jax
import jax.numpy as jnp
from jax import lax
from jax.experimental import pallas as pl
from jax.experimental.pallas import tpu as pltpu
from jax.experimental.pallas import tpu_sc as plsc

N = 100000
E = 1600000
LANE = 16
N_SC = 2
N_TILES = 16
BLK = 512
ROW = 128
NDMA = BLK // ROW
EPT = -(-E // (N_TILES * BLK)) * BLK
E_PAD = EPT * N_TILES
NPT = 6256
NPT_LAST = N - 15 * NPT
RND = 512
NRND = 13
ACC_R = N + 16
NBLK = EPT // BLK
IDXROWS = E_PAD // ROW

_MESH = plsc.VectorSubcoreMesh(core_axis_name="c", subcore_axis_name="s")


def _build_widx(widx, base_node, nc, ch, iota_nc):
    for k in range(RND // ROW):
        for q in range(ROW // LANE):
            off = (base_node + k * ROW + q * LANE) * nc + ch
            widx[k, pl.ds(q * LANE, LANE)] = iota_nc + off


def _round_base(s, m):
    last = jnp.where(s == N_TILES - 1, NPT_LAST - RND, NPT - RND)
    off = jnp.where(m == NRND - 1, last, m * RND)
    return pl.multiple_of(s * NPT + off, 16)


def _agg_body(nc, gflat, src2, dst2, oflat, acc,
              sidxA, sidxB, gidxA, gidxB, didx0, didx1, didx2, didx3,
              rowsA, rowsB, widx, bounce, gsem, ssem, isem):
    c = lax.axis_index("c")
    s = lax.axis_index("s")
    nch = nc // N_SC
    iota_nc = lax.iota(jnp.int32, LANE) * nc
    sidx2 = (sidxA, sidxB)
    gidx2 = (gidxA, gidxB)
    didx4 = (didx0, didx1, didx2, didx3)
    rows2 = (rowsA, rowsB)

    def idx_row(jb):
        return jnp.minimum(s * (EPT // ROW) + jb * NDMA, IDXROWS - NDMA)

    def fire_idx(jb, sx, dx):
        r = idx_row(jb)
        pltpu.async_copy(src2.at[pl.ds(r, NDMA)], sx, isem)
        pltpu.async_copy(dst2.at[pl.ds(r, NDMA)], dx, isem)

    def wait_idx(sx, dx):
        pltpu.make_async_copy(src2.at[pl.ds(0, NDMA)], sx, isem).wait()
        pltpu.make_async_copy(dst2.at[pl.ds(0, NDMA)], dx, isem).wait()

    def compute_gidx(gx, sx, ch):
        for k in range(NDMA):
            for q in range(ROW // LANE):
                sl = pl.ds(q * LANE, LANE)
                gx[k, sl] = sx[k, sl] * nc + ch

    def fire_gathers(gx, rx):
        for k in range(NDMA):
            pltpu.async_copy(gflat.at[gx.at[k]],
                             rx.at[pl.ds(k * ROW, ROW)], gsem)

    def wait_gathers(gx, rx):
        for k in range(NDMA):
            pltpu.make_async_copy(gflat.at[gx.at[k]],
                                  rx.at[pl.ds(k * ROW, ROW)], gsem).wait()

    def fire_scatter(rx, dx):
        for k in range(NDMA):
            pltpu.async_copy(rx.at[pl.ds(k * ROW, ROW)],
                             acc.at[dx.at[k]], ssem, add=True)

    def wait_scatter(rx, dx):
        for k in range(NDMA):
            pltpu.make_async_copy(rx.at[pl.ds(k * ROW, ROW)],
                                  acc.at[dx.at[k]], ssem).wait()

    def chunk(j, carry):
        ch = j * N_SC + c
        def init_round(m, carry2):
            b = _round_base(s, m)
            _build_widx(widx, b, nc, ch, iota_nc)
            cps = [
                pltpu.async_copy(gflat.at[widx.at[k]],
                                 bounce.at[pl.ds(k * ROW, ROW)], gsem)
                for k in range(RND // ROW)
            ]
            for cp in cps:
                cp.wait()
            pltpu.sync_copy(bounce, acc.at[pl.ds(b, RND)])
            return carry2

        lax.fori_loop(0, NRND, init_round, 0)
        plsc.subcore_barrier()

        pltpu.sync_copy(src2.at[pl.ds(idx_row(0), NDMA)], sidxA)
        pltpu.sync_copy(dst2.at[pl.ds(idx_row(0), NDMA)], didx0)
        compute_gidx(gidxA, sidxA, ch)
        fire_gathers(gidxA, rowsA)
        fire_idx(1, sidxB, didx1)
        fire_idx(2, sidxA, didx2)

        def quad(i, carry2):
            jb = i * 4
            for t in range(4):
                X, Y = rows2[t % 2], rows2[(t + 1) % 2]
                gX, gY = gidx2[t % 2], gidx2[(t + 1) % 2]
                sY = sidx2[(t + 1) % 2]
                wait_idx(sY, didx4[(t + 1) % 4])
                compute_gidx(gY, sY, ch)
                if t == 0:
                    @pl.when(i > 0)
                    def _():
                        wait_scatter(Y, didx4[3])
                else:
                    wait_scatter(Y, didx4[(t + 3) % 4])
                fire_gathers(gY, Y)
                wait_gathers(gX, X)
                fire_scatter(X, didx4[t % 4])
                fire_idx(jb + t + 3,
                         sidx2[(t + 3) % 2], didx4[(t + 3) % 4])
            return carry2

        lax.fori_loop(0, NBLK // 4, quad, 0)
        wait_gathers(gidx2[NBLK % 2], rows2[NBLK % 2])
        wait_scatter(rows2[(NBLK - 1) % 2], didx4[(NBLK - 1) % 4])
        wait_idx(sidx2[(NBLK + 1) % 2], didx4[(NBLK + 1) % 4])
        wait_idx(sidx2[NBLK % 2], didx4[(NBLK + 2) % 4])
        plsc.subcore_barrier()

        def wout_round(m, carry2):
            b = _round_base(s, m)
            _build_widx(widx, b, nc, ch, iota_nc)
            pltpu.sync_copy(acc.at[pl.ds(b, RND)], bounce)
            cps = [
                pltpu.async_copy(bounce.at[pl.ds(k * ROW, ROW)],
                                 oflat.at[widx.at[k]], gsem)
                for k in range(RND // ROW)
            ]
            for cp in cps:
                cp.wait()
            return carry2

        lax.fori_loop(0, NRND, wout_round, 0)
        plsc.subcore_barrier()
        return carry

    lax.fori_loop(0, nch, chunk, 0)


@functools.cache
def _make_agg(nc):
    idx_t = pltpu.VMEM((NDMA, ROW), jnp.int32)
    rows_t = pltpu.VMEM((BLK, LANE), jnp.float32)
    return pl.kernel(
        functools.partial(_agg_body, nc),
        out_type=jax.ShapeDtypeStruct((N * nc, LANE), jnp.float32),
        mesh=_MESH,
        scratch_types=[
            pltpu.VMEM_SHARED((ACC_R, LANE), jnp.float32),
            idx_t, idx_t, idx_t, idx_t,
            idx_t, idx_t, idx_t, idx_t,
            rows_t, rows_t,
            pltpu.VMEM((RND // ROW, ROW), jnp.int32),
            pltpu.VMEM((RND, LANE), jnp.float32),
            pltpu.SemaphoreType.DMA,
            pltpu.SemaphoreType.DMA,
            pltpu.SemaphoreType.DMA,
        ],
        compiler_params=pltpu.CompilerParams(use_tc_tiling_on_sc=False),
    )


def _deg_body(dst2, out2, acc, didx, buf):
    c = lax.axis_index("c")
    s = lax.axis_index("s")
    r0 = s * NPT
    ones = jnp.full((LANE,), 1.0, jnp.float32)
    for k in range(ROW):
        buf[k, :] = ones

    @pl.when(c == 0)
    def _init():
        for m in range(48):
            pltpu.sync_copy(buf, acc.at[pl.ds(r0 + m * ROW, ROW)])
        tail = r0 + pl.multiple_of(
            jnp.where(s == N_TILES - 1, NPT_LAST - ROW, NPT - ROW), 16)
        pltpu.sync_copy(buf, acc.at[pl.ds(tail, ROW)])
    plsc.subcore_barrier()

    @pl.when(c == 0)
    def _scatter():
        def blk(ib, carry):
            row = s * (EPT // ROW) + ib * NDMA
            pltpu.sync_copy(dst2.at[pl.ds(row, NDMA)], didx)
            for k in range(NDMA):
                pltpu.sync_copy(buf, acc.at[didx.at[k]], add=True)
            return carry
        lax.fori_loop(0, EPT // BLK, blk, 0)
    plsc.subcore_barrier()

    @pl.when(c == 0)
    def _wout():
        @pl.when(s < N_TILES - 1)
        def _a():
            pltpu.sync_copy(acc.at[pl.ds(r0, NPT)], out2.at[pl.ds(r0, NPT)])
        @pl.when(s == N_TILES - 1)
        def _b():
            pltpu.sync_copy(acc.at[pl.ds(r0, NPT_LAST)],
                            out2.at[pl.ds(r0, NPT_LAST)])


_deg_call = pl.kernel(
    _deg_body,
    out_type=jax.ShapeDtypeStruct((N, LANE), jnp.float32),
    mesh=_MESH,
    scratch_types=[
        pltpu.VMEM_SHARED((ACC_R, LANE), jnp.float32),
        pltpu.VMEM((NDMA, ROW), jnp.int32),
        pltpu.VMEM((ROW, LANE), jnp.float32),
    ],
    compiler_params=pltpu.CompilerParams(use_tc_tiling_on_sc=False),
)


BN = 2000
GRID = N // BN
_PREC = lax.Precision.HIGHEST


def _dinv(deg_ref):
    return lax.rsqrt(deg_ref[:, 0:1])


def _g0_body(x_ref, deg_ref, o_ref):
    o_ref[...] = x_ref[...] * _dinv(deg_ref)


def _post_body(A_ref, deg_ref, W_ref, b_ref, o_ref):
    dinv = _dinv(deg_ref)
    u = A_ref[...] * dinv
    y = jnp.dot(u, W_ref[...], preferred_element_type=jnp.float32,
                precision=_PREC) + b_ref[0:1, :]
    o_ref[...] = jnp.maximum(y, 0.0) * dinv


def _post4_body(A_ref, deg_ref, W4_ref, b4_ref, W5_ref, o_ref):
    dinv = _dinv(deg_ref)
    u = A_ref[...] * dinv
    y = jnp.dot(u, W4_ref[...], preferred_element_type=jnp.float32,
                precision=_PREC) + b4_ref[0:1, :]
    g4 = jnp.maximum(y, 0.0) * dinv
    o_ref[...] = jnp.dot(g4, W5_ref[...], preferred_element_type=jnp.float32,
                         precision=_PREC)


def _final_body(A_ref, deg_ref, b_ref, o_ref):
    o_ref[...] = A_ref[...] * _dinv(deg_ref) + b_ref[0:1, :]


def _row_spec(d):
    return pl.BlockSpec((BN, d), lambda i: (i, 0))


def _full_spec(r, c):
    return pl.BlockSpec((r, c), lambda i: (0, 0))


def _b2d(b):
    return jnp.broadcast_to(b.reshape(1, -1), (8, b.shape[0]))


def _tc_g0(x, deg16):
    return pl.pallas_call(
        _g0_body, grid=(GRID,),
        in_specs=[_row_spec(x.shape[1]), _row_spec(LANE)],
        out_specs=_row_spec(x.shape[1]),
        out_shape=jax.ShapeDtypeStruct((N, x.shape[1]), jnp.float32),
    )(x, deg16)


def _tc_post(A, deg16, W, b):
    din, dout = W.shape
    return pl.pallas_call(
        _post_body, grid=(GRID,),
        in_specs=[_row_spec(din), _row_spec(LANE),
                  _full_spec(din, dout), _full_spec(8, dout)],
        out_specs=_row_spec(dout),
        out_shape=jax.ShapeDtypeStruct((N, dout), jnp.float32),
    )(A, deg16, W, _b2d(b))


def _tc_post4(A, deg16, W4, b4, W5):
    return pl.pallas_call(
        _post4_body, grid=(GRID,),
        in_specs=[_row_spec(256), _row_spec(LANE), _full_spec(256, 512),
                  _full_spec(8, 512), _full_spec(512, 128)],
        out_specs=_row_spec(128),
        out_shape=jax.ShapeDtypeStruct((N, 128), jnp.float32),
    )(A, deg16, W4, _b2d(b4), W5)


def _tc_final(A, deg16, b):
    return pl.pallas_call(
        _final_body, grid=(GRID,),
        in_specs=[_row_spec(128), _row_spec(LANE), _full_spec(8, 128)],
        out_specs=_row_spec(128),
        out_shape=jax.ShapeDtypeStruct((N, 128), jnp.float32),
    )(A, deg16, _b2d(b))


def kernel(x, edge_index, W1, b1, W2, b2, W3, b3, W4, b4, W5, b5):
    src = edge_index[0].astype(jnp.int32)
    dst = edge_index[1].astype(jnp.int32)
    pad = E_PAD - E
    src2 = jnp.concatenate([src, jnp.zeros((pad,), jnp.int32)]
                           ).reshape(E_PAD // ROW, ROW)
    dst2 = jnp.concatenate([dst, jnp.full((pad,), N, jnp.int32)]
                           ).reshape(E_PAD // ROW, ROW)

    def agg(g, nc):
        gflat = g.reshape(N * nc, LANE)
        return _make_agg(nc)(gflat, src2, dst2).reshape(N, nc * LANE)

    deg16 = _deg_call(dst2)
    g = _tc_g0(x, deg16)
    g = _tc_post(agg(g, 2), deg16, W1, b1)
    g = _tc_post(agg(g, 4), deg16, W2, b2)
    g = _tc_post(agg(g, 8), deg16, W3, b3)
    z = _tc_post4(agg(g, 16), deg16, W4, b4, W5)
    return _tc_final(agg(z, 8), deg16, b5)


# speedup vs baseline: 11.6477x; 1.0335x over previous
import functools

import jax
import jax.numpy as jnp
from jax import lax
from jax.experimental import pallas as pl
from jax.experimental.pallas import tpu as pltpu
from jax.experimental.pallas import tpu_sc as plsc

N = 100000
E = 1600000
LANE = 16
N_SC = 2
N_TILES = 16
BLK = 512
ROW = 128
NDMA = BLK // ROW
EPT = -(-E // (N_TILES * BLK)) * BLK
E_PAD = EPT * N_TILES
NPT = 6256
NPT_LAST = N - 15 * NPT
RND = 512
NRND = 13
ACC_R = N + 16
NBLK = EPT // BLK
IDXROWS = E_PAD // ROW

_MESH = plsc.VectorSubcoreMesh(core_axis_name="c", subcore_axis_name="s")


def _build_widx(widx, base_node, nc, ch, iota_nc):
    for k in range(RND // ROW):
        for q in range(ROW // LANE):
            off = (base_node + k * ROW + q * LANE) * nc + ch
            widx[k, pl.ds(q * LANE, LANE)] = iota_nc + off


def _round_base(s, m):
    last = jnp.where(s == N_TILES - 1, NPT_LAST - RND, NPT - RND)
    off = jnp.where(m == NRND - 1, last, m * RND)
    return pl.multiple_of(s * NPT + off, 16)


def _agg_body(nc, gflat, src2, dst2, zrnd, oflat, acc,
              sidxA, sidxB, gidxA, gidxB, didx0, didx1, didx2, didx3,
              rowsA, rowsB, widx, bounce, gsem, ssem, isem):
    c = lax.axis_index("c")
    s = lax.axis_index("s")
    nch = nc // N_SC
    iota_nc = lax.iota(jnp.int32, LANE) * nc
    sidx2 = (sidxA, sidxB)
    gidx2 = (gidxA, gidxB)
    didx4 = (didx0, didx1, didx2, didx3)
    rows2 = (rowsA, rowsB)

    def idx_row(jb):
        return jnp.minimum(s * (EPT // ROW) + jb * NDMA, IDXROWS - NDMA)

    def fire_idx(jb, sx, dx):
        r = idx_row(jb)
        pltpu.async_copy(src2.at[pl.ds(r, NDMA)], sx, isem)
        pltpu.async_copy(dst2.at[pl.ds(r, NDMA)], dx, isem)

    def wait_idx(sx, dx):
        pltpu.make_async_copy(src2.at[pl.ds(0, NDMA)], sx, isem).wait()
        pltpu.make_async_copy(dst2.at[pl.ds(0, NDMA)], dx, isem).wait()

    def compute_gidx(gx, sx, ch):
        for k in range(NDMA):
            for q in range(ROW // LANE):
                sl = pl.ds(q * LANE, LANE)
                gx[k, sl] = sx[k, sl] * nc + ch

    def fire_gathers(gx, rx):
        for k in range(NDMA):
            pltpu.async_copy(gflat.at[gx.at[k]],
                             rx.at[pl.ds(k * ROW, ROW)], gsem)

    def wait_gathers(gx, rx):
        for k in range(NDMA):
            pltpu.make_async_copy(gflat.at[gx.at[k]],
                                  rx.at[pl.ds(k * ROW, ROW)], gsem).wait()

    def fire_scatter(rx, dx):
        for k in range(NDMA):
            pltpu.async_copy(rx.at[pl.ds(k * ROW, ROW)],
                             acc.at[dx.at[k]], ssem, add=True)

    def wait_scatter(rx, dx):
        for k in range(NDMA):
            pltpu.make_async_copy(rx.at[pl.ds(k * ROW, ROW)],
                                  acc.at[dx.at[k]], ssem).wait()

    def chunk(j, carry):
        ch = j * N_SC + c
        pltpu.sync_copy(zrnd, bounce)

        def zero_round(m, carry2):
            pltpu.sync_copy(bounce, acc.at[pl.ds(_round_base(s, m), RND)])
            return carry2

        lax.fori_loop(0, NRND, zero_round, 0)
        plsc.subcore_barrier()

        pltpu.sync_copy(src2.at[pl.ds(idx_row(0), NDMA)], sidxA)
        pltpu.sync_copy(dst2.at[pl.ds(idx_row(0), NDMA)], didx0)
        compute_gidx(gidxA, sidxA, ch)
        fire_gathers(gidxA, rowsA)
        fire_idx(1, sidxB, didx1)
        fire_idx(2, sidxA, didx2)

        def quad(i, carry2):
            jb = i * 4
            for t in range(4):
                X, Y = rows2[t % 2], rows2[(t + 1) % 2]
                gX, gY = gidx2[t % 2], gidx2[(t + 1) % 2]
                sY = sidx2[(t + 1) % 2]
                wait_idx(sY, didx4[(t + 1) % 4])
                compute_gidx(gY, sY, ch)
                if t == 0:
                    @pl.when(i > 0)
                    def _():
                        wait_scatter(Y, didx4[3])
                else:
                    wait_scatter(Y, didx4[(t + 3) % 4])
                fire_gathers(gY, Y)
                wait_gathers(gX, X)
                fire_scatter(X, didx4[t % 4])
                fire_idx(jb + t + 3,
                         sidx2[(t + 3) % 2], didx4[(t + 3) % 4])
            return carry2

        lax.fori_loop(0, NBLK // 4, quad, 0)
        wait_gathers(gidx2[NBLK % 2], rows2[NBLK % 2])
        wait_scatter(rows2[(NBLK - 1) % 2], didx4[(NBLK - 1) % 4])
        wait_idx(sidx2[(NBLK + 1) % 2], didx4[(NBLK + 1) % 4])
        wait_idx(sidx2[NBLK % 2], didx4[(NBLK + 2) % 4])
        plsc.subcore_barrier()

        def wout_round(m, carry2):
            b = _round_base(s, m)
            _build_widx(widx, b, nc, ch, iota_nc)
            pltpu.sync_copy(acc.at[pl.ds(b, RND)], bounce)
            cps = [
                pltpu.async_copy(bounce.at[pl.ds(k * ROW, ROW)],
                                 oflat.at[widx.at[k]], gsem)
                for k in range(RND // ROW)
            ]
            for cp in cps:
                cp.wait()
            return carry2

        lax.fori_loop(0, NRND, wout_round, 0)
        plsc.subcore_barrier()
        return carry

    lax.fori_loop(0, nch, chunk, 0)


@functools.cache
def _make_agg(nc):
    idx_t = pltpu.VMEM((NDMA, ROW), jnp.int32)
    rows_t = pltpu.VMEM((BLK, LANE), jnp.float32)
    return pl.kernel(
        functools.partial(_agg_body, nc),
        out_type=jax.ShapeDtypeStruct((N * nc, LANE), jnp.float32),
        mesh=_MESH,
        scratch_types=[
            pltpu.VMEM_SHARED((ACC_R, LANE), jnp.float32),
            idx_t, idx_t, idx_t, idx_t,
            idx_t, idx_t, idx_t, idx_t,
            rows_t, rows_t,
            pltpu.VMEM((RND // ROW, ROW), jnp.int32),
            pltpu.VMEM((RND, LANE), jnp.float32),
            pltpu.SemaphoreType.DMA,
            pltpu.SemaphoreType.DMA,
            pltpu.SemaphoreType.DMA,
        ],
        compiler_params=pltpu.CompilerParams(use_tc_tiling_on_sc=False),
    )


def _deg_body(dst2, out2, acc, didx, buf):
    c = lax.axis_index("c")
    s = lax.axis_index("s")
    r0 = s * NPT
    ones = jnp.full((LANE,), 1.0, jnp.float32)
    for k in range(ROW):
        buf[k, :] = ones

    @pl.when(c == 0)
    def _init():
        for m in range(48):
            pltpu.sync_copy(buf, acc.at[pl.ds(r0 + m * ROW, ROW)])
        tail = r0 + pl.multiple_of(
            jnp.where(s == N_TILES - 1, NPT_LAST - ROW, NPT - ROW), 16)
        pltpu.sync_copy(buf, acc.at[pl.ds(tail, ROW)])
    plsc.subcore_barrier()

    @pl.when(c == 0)
    def _scatter():
        def blk(ib, carry):
            row = s * (EPT // ROW) + ib * NDMA
            pltpu.sync_copy(dst2.at[pl.ds(row, NDMA)], didx)
            for k in range(NDMA):
                pltpu.sync_copy(buf, acc.at[didx.at[k]], add=True)
            return carry
        lax.fori_loop(0, EPT // BLK, blk, 0)
    plsc.subcore_barrier()

    @pl.when(c == 0)
    def _wout():
        @pl.when(s < N_TILES - 1)
        def _a():
            pltpu.sync_copy(acc.at[pl.ds(r0, NPT)], out2.at[pl.ds(r0, NPT)])
        @pl.when(s == N_TILES - 1)
        def _b():
            pltpu.sync_copy(acc.at[pl.ds(r0, NPT_LAST)],
                            out2.at[pl.ds(r0, NPT_LAST)])


_deg_call = pl.kernel(
    _deg_body,
    out_type=jax.ShapeDtypeStruct((N, LANE), jnp.float32),
    mesh=_MESH,
    scratch_types=[
        pltpu.VMEM_SHARED((ACC_R, LANE), jnp.float32),
        pltpu.VMEM((NDMA, ROW), jnp.int32),
        pltpu.VMEM((ROW, LANE), jnp.float32),
    ],
    compiler_params=pltpu.CompilerParams(use_tc_tiling_on_sc=False),
)


BN = 2000
GRID = N // BN
_PREC = lax.Precision.HIGHEST


def _dinv(deg_ref):
    return lax.rsqrt(deg_ref[:, 0:1])


def _g0_body(x_ref, deg_ref, o_ref):
    o_ref[...] = x_ref[...] * _dinv(deg_ref)


def _post_body(A_ref, g_ref, deg_ref, W_ref, b_ref, o_ref):
    dinv = _dinv(deg_ref)
    u = (A_ref[...] + g_ref[...]) * dinv
    y = jnp.dot(u, W_ref[...], preferred_element_type=jnp.float32,
                precision=_PREC) + b_ref[0:1, :]
    o_ref[...] = jnp.maximum(y, 0.0) * dinv


def _post4_body(A_ref, g_ref, deg_ref, W4_ref, b4_ref, W5_ref, o_ref):
    dinv = _dinv(deg_ref)
    u = (A_ref[...] + g_ref[...]) * dinv
    y = jnp.dot(u, W4_ref[...], preferred_element_type=jnp.float32,
                precision=_PREC) + b4_ref[0:1, :]
    g4 = jnp.maximum(y, 0.0) * dinv
    o_ref[...] = jnp.dot(g4, W5_ref[...], preferred_element_type=jnp.float32,
                         precision=_PREC)


def _final_body(A_ref, g_ref, deg_ref, b_ref, o_ref):
    o_ref[...] = (A_ref[...] + g_ref[...]) * _dinv(deg_ref) + b_ref[0:1, :]


def _row_spec(d):
    return pl.BlockSpec((BN, d), lambda i: (i, 0))


def _full_spec(r, c):
    return pl.BlockSpec((r, c), lambda i: (0, 0))


def _b2d(b):
    return jnp.broadcast_to(b.reshape(1, -1), (8, b.shape[0]))


def _tc_g0(x, deg16):
    return pl.pallas_call(
        _g0_body, grid=(GRID,),
        in_specs=[_row_spec(x.shape[1]), _row_spec(LANE)],
        out_specs=_row_spec(x.shape[1]),
        out_shape=jax.ShapeDtypeStruct((N, x.shape[1]), jnp.float32),
    )(x, deg16)


def _tc_post(A, g, deg16, W, b):
    din, dout = W.shape
    return pl.pallas_call(
        _post_body, grid=(GRID,),
        in_specs=[_row_spec(din), _row_spec(din), _row_spec(LANE),
                  _full_spec(din, dout), _full_spec(8, dout)],
        out_specs=_row_spec(dout),
        out_shape=jax.ShapeDtypeStruct((N, dout), jnp.float32),
    )(A, g, deg16, W, _b2d(b))


def _tc_post4(A, g, deg16, W4, b4, W5):
    return pl.pallas_call(
        _post4_body, grid=(GRID,),
        in_specs=[_row_spec(256), _row_spec(256), _row_spec(LANE),
                  _full_spec(256, 512), _full_spec(8, 512),
                  _full_spec(512, 128)],
        out_specs=_row_spec(128),
        out_shape=jax.ShapeDtypeStruct((N, 128), jnp.float32),
    )(A, g, deg16, W4, _b2d(b4), W5)


def _tc_final(A, z, deg16, b):
    return pl.pallas_call(
        _final_body, grid=(GRID,),
        in_specs=[_row_spec(128), _row_spec(128), _row_spec(LANE),
                  _full_spec(8, 128)],
        out_specs=_row_spec(128),
        out_shape=jax.ShapeDtypeStruct((N, 128), jnp.float32),
    )(A, z, deg16, _b2d(b))


def kernel(x, edge_index, W1, b1, W2, b2, W3, b3, W4, b4, W5, b5):
    src = edge_index[0].astype(jnp.int32)
    dst = edge_index[1].astype(jnp.int32)
    pad = E_PAD - E
    src2 = jnp.concatenate([src, jnp.zeros((pad,), jnp.int32)]
                           ).reshape(E_PAD // ROW, ROW)
    dst2 = jnp.concatenate([dst, jnp.full((pad,), N, jnp.int32)]
                           ).reshape(E_PAD // ROW, ROW)

    zrnd = jnp.zeros((RND, LANE), jnp.float32)

    def agg(g, nc):
        gflat = g.reshape(N * nc, LANE)
        return _make_agg(nc)(gflat, src2, dst2, zrnd).reshape(N, nc * LANE)

    deg16 = _deg_call(dst2)
    g = _tc_g0(x, deg16)
    g = _tc_post(agg(g, 2), g, deg16, W1, b1)
    g = _tc_post(agg(g, 4), g, deg16, W2, b2)
    g = _tc_post(agg(g, 8), g, deg16, W3, b3)
    z = _tc_post4(agg(g, 16), g, deg16, W4, b4, W5)
    return _tc_final(agg(z, 8), z, deg16, b5)


# speedup vs baseline: 12.4513x; 1.0690x over previous
import functools

import jax
import jax.numpy as jnp
from jax import lax
from jax.experimental import pallas as pl
from jax.experimental.pallas import tpu as pltpu
from jax.experimental.pallas import tpu_sc as plsc

N = 100000
E = 1600000
LANE = 16
N_SC = 2
N_TILES = 16
BLK = 512
ROW = 128
NDMA = BLK // ROW
EPT = -(-E // (N_TILES * BLK)) * BLK
E_PAD = EPT * N_TILES
NPT = 6256
NPT_LAST = N - 15 * NPT
RND = 512
NRND = 13
ACC_R = N + 16
NBLK = EPT // BLK
IDXROWS = E_PAD // ROW

_MESH = plsc.VectorSubcoreMesh(core_axis_name="c", subcore_axis_name="s")


def _build_widx(widx, base_node, nc, ch, iota_nc):
    for k in range(RND // ROW):
        for q in range(ROW // LANE):
            off = (base_node + k * ROW + q * LANE) * nc + ch
            widx[k, pl.ds(q * LANE, LANE)] = iota_nc + off


def _round_base(s, m):
    last = jnp.where(s == N_TILES - 1, NPT_LAST - RND, NPT - RND)
    off = jnp.where(m == NRND - 1, last, m * RND)
    return pl.multiple_of(s * NPT + off, 16)


def _agg_body(nc, gflat, src2, dst2, zrnd, oflat, acc,
              sidxA, sidxB, gidxA, gidxB, didx0, didx1, didx2, didx3,
              rowsA, rowsB, widx, bounce, gsem, ssem, isem):
    c = lax.axis_index("c")
    s = lax.axis_index("s")
    nch = nc // N_SC
    iota_nc = lax.iota(jnp.int32, LANE) * nc
    sidx2 = (sidxA, sidxB)
    gidx2 = (gidxA, gidxB)
    didx4 = (didx0, didx1, didx2, didx3)
    rows2 = (rowsA, rowsB)

    def idx_row(jb):
        return jnp.minimum(s * (EPT // ROW) + jb * NDMA, IDXROWS - NDMA)

    def fire_idx(jb, sx, dx):
        r = idx_row(jb)
        pltpu.async_copy(src2.at[pl.ds(r, NDMA)], sx, isem)
        pltpu.async_copy(dst2.at[pl.ds(r, NDMA)], dx, isem)

    def wait_idx(sx, dx):
        pltpu.make_async_copy(src2.at[pl.ds(0, NDMA)], sx, isem).wait()
        pltpu.make_async_copy(dst2.at[pl.ds(0, NDMA)], dx, isem).wait()

    def compute_gidx(gx, sx, ch):
        for k in range(NDMA):
            for q in range(ROW // LANE):
                sl = pl.ds(q * LANE, LANE)
                gx[k, sl] = sx[k, sl] * nc + ch

    def fire_gathers(gx, rx):
        for k in range(NDMA):
            pltpu.async_copy(gflat.at[gx.at[k]],
                             rx.at[pl.ds(k * ROW, ROW)], gsem)

    def wait_gathers(gx, rx):
        for k in range(NDMA):
            pltpu.make_async_copy(gflat.at[gx.at[k]],
                                  rx.at[pl.ds(k * ROW, ROW)], gsem).wait()

    def fire_scatter(rx, dx):
        for k in range(NDMA):
            pltpu.async_copy(rx.at[pl.ds(k * ROW, ROW)],
                             acc.at[dx.at[k]], ssem, add=True)

    def wait_scatter(rx, dx):
        for k in range(NDMA):
            pltpu.make_async_copy(rx.at[pl.ds(k * ROW, ROW)],
                                  acc.at[dx.at[k]], ssem).wait()

    def chunk(j, carry):
        ch = j * N_SC + c
        pltpu.sync_copy(zrnd, bounce)

        def zero_round(m, carry2):
            pltpu.sync_copy(bounce, acc.at[pl.ds(_round_base(s, m), RND)])
            return carry2

        lax.fori_loop(0, NRND, zero_round, 0)
        plsc.subcore_barrier()

        pltpu.sync_copy(src2.at[pl.ds(idx_row(0), NDMA)], sidxA)
        pltpu.sync_copy(dst2.at[pl.ds(idx_row(0), NDMA)], didx0)
        compute_gidx(gidxA, sidxA, ch)
        fire_gathers(gidxA, rowsA)
        fire_idx(1, sidxB, didx1)
        fire_idx(2, sidxA, didx2)

        def quad(i, carry2):
            jb = i * 4
            for t in range(4):
                X, Y = rows2[t % 2], rows2[(t + 1) % 2]
                gX, gY = gidx2[t % 2], gidx2[(t + 1) % 2]
                sY = sidx2[(t + 1) % 2]
                wait_idx(sY, didx4[(t + 1) % 4])
                compute_gidx(gY, sY, ch)
                if t == 0:
                    @pl.when(i > 0)
                    def _():
                        wait_scatter(Y, didx4[3])
                else:
                    wait_scatter(Y, didx4[(t + 3) % 4])
                fire_gathers(gY, Y)
                wait_gathers(gX, X)
                fire_scatter(X, didx4[t % 4])
                fire_idx(jb + t + 3,
                         sidx2[(t + 3) % 2], didx4[(t + 3) % 4])
            return carry2

        lax.fori_loop(0, NBLK // 4, quad, 0)
        wait_gathers(gidx2[NBLK % 2], rows2[NBLK % 2])
        wait_scatter(rows2[(NBLK - 1) % 2], didx4[(NBLK - 1) % 4])
        wait_idx(sidx2[(NBLK + 1) % 2], didx4[(NBLK + 1) % 4])
        wait_idx(sidx2[NBLK % 2], didx4[(NBLK + 2) % 4])
        plsc.subcore_barrier()

        def wout_round(m, carry2):
            b = _round_base(s, m)
            _build_widx(widx, b, nc, ch, iota_nc)
            pltpu.sync_copy(acc.at[pl.ds(b, RND)], bounce)
            cps = [
                pltpu.async_copy(bounce.at[pl.ds(k * ROW, ROW)],
                                 oflat.at[widx.at[k]], gsem)
                for k in range(RND // ROW)
            ]
            for cp in cps:
                cp.wait()
            return carry2

        lax.fori_loop(0, NRND, wout_round, 0)
        plsc.subcore_barrier()
        return carry

    lax.fori_loop(0, nch, chunk, 0)


@functools.cache
def _make_agg(nc):
    idx_t = pltpu.VMEM((NDMA, ROW), jnp.int32)
    rows_t = pltpu.VMEM((BLK, LANE), jnp.float32)
    return pl.kernel(
        functools.partial(_agg_body, nc),
        out_type=jax.ShapeDtypeStruct((N * nc, LANE), jnp.float32),
        mesh=_MESH,
        scratch_types=[
            pltpu.VMEM_SHARED((ACC_R, LANE), jnp.float32),
            idx_t, idx_t, idx_t, idx_t,
            idx_t, idx_t, idx_t, idx_t,
            rows_t, rows_t,
            pltpu.VMEM((RND // ROW, ROW), jnp.int32),
            pltpu.VMEM((RND, LANE), jnp.float32),
            pltpu.SemaphoreType.DMA,
            pltpu.SemaphoreType.DMA,
            pltpu.SemaphoreType.DMA,
        ],
        compiler_params=pltpu.CompilerParams(use_tc_tiling_on_sc=False),
    )


def _deg_body(dst2, out2, acc, didx, buf):
    c = lax.axis_index("c")
    s = lax.axis_index("s")
    r0 = s * NPT
    ones = jnp.full((LANE,), 1.0, jnp.float32)
    for k in range(ROW):
        buf[k, :] = ones

    @pl.when(c == 0)
    def _init():
        for m in range(48):
            pltpu.sync_copy(buf, acc.at[pl.ds(r0 + m * ROW, ROW)])
        tail = r0 + pl.multiple_of(
            jnp.where(s == N_TILES - 1, NPT_LAST - ROW, NPT - ROW), 16)
        pltpu.sync_copy(buf, acc.at[pl.ds(tail, ROW)])
    plsc.subcore_barrier()

    @pl.when(c == 0)
    def _scatter():
        def blk(ib, carry):
            row = s * (EPT // ROW) + ib * NDMA
            pltpu.sync_copy(dst2.at[pl.ds(row, NDMA)], didx)
            for k in range(NDMA):
                pltpu.sync_copy(buf, acc.at[didx.at[k]], add=True)
            return carry
        lax.fori_loop(0, EPT // BLK, blk, 0)
    plsc.subcore_barrier()

    @pl.when(c == 0)
    def _wout():
        @pl.when(s < N_TILES - 1)
        def _a():
            pltpu.sync_copy(acc.at[pl.ds(r0, NPT)], out2.at[pl.ds(r0, NPT)])
        @pl.when(s == N_TILES - 1)
        def _b():
            pltpu.sync_copy(acc.at[pl.ds(r0, NPT_LAST)],
                            out2.at[pl.ds(r0, NPT_LAST)])


_deg_call = pl.kernel(
    _deg_body,
    out_type=jax.ShapeDtypeStruct((N, LANE), jnp.float32),
    mesh=_MESH,
    scratch_types=[
        pltpu.VMEM_SHARED((ACC_R, LANE), jnp.float32),
        pltpu.VMEM((NDMA, ROW), jnp.int32),
        pltpu.VMEM((ROW, LANE), jnp.float32),
    ],
    compiler_params=pltpu.CompilerParams(use_tc_tiling_on_sc=False),
)


BN = 2000
GRID = N // BN
_PREC = lax.Precision.DEFAULT


def _dinv(deg_ref):
    return lax.rsqrt(deg_ref[:, 0:1])


def _g0_body(x_ref, deg_ref, o_ref):
    o_ref[...] = x_ref[...] * _dinv(deg_ref)


def _post_body(A_ref, g_ref, deg_ref, W_ref, b_ref, o_ref):
    dinv = _dinv(deg_ref)
    u = (A_ref[...] + g_ref[...]) * dinv
    y = jnp.dot(u, W_ref[...], preferred_element_type=jnp.float32,
                precision=_PREC) + b_ref[0:1, :]
    o_ref[...] = jnp.maximum(y, 0.0) * dinv


def _post4_body(A_ref, g_ref, deg_ref, W4_ref, b4_ref, W5_ref, o_ref):
    dinv = _dinv(deg_ref)
    u = (A_ref[...] + g_ref[...]) * dinv
    y = jnp.dot(u, W4_ref[...], preferred_element_type=jnp.float32,
                precision=_PREC) + b4_ref[0:1, :]
    g4 = jnp.maximum(y, 0.0) * dinv
    o_ref[...] = jnp.dot(g4, W5_ref[...], preferred_element_type=jnp.float32,
                         precision=_PREC)


def _final_body(A_ref, g_ref, deg_ref, b_ref, o_ref):
    o_ref[...] = (A_ref[...] + g_ref[...]) * _dinv(deg_ref) + b_ref[0:1, :]


def _row_spec(d):
    return pl.BlockSpec((BN, d), lambda i: (i, 0))


def _full_spec(r, c):
    return pl.BlockSpec((r, c), lambda i: (0, 0))


def _b2d(b):
    return jnp.broadcast_to(b.reshape(1, -1), (8, b.shape[0]))


def _tc_g0(x, deg16):
    return pl.pallas_call(
        _g0_body, grid=(GRID,),
        in_specs=[_row_spec(x.shape[1]), _row_spec(LANE)],
        out_specs=_row_spec(x.shape[1]),
        out_shape=jax.ShapeDtypeStruct((N, x.shape[1]), jnp.float32),
    )(x, deg16)


def _tc_post(A, g, deg16, W, b):
    din, dout = W.shape
    return pl.pallas_call(
        _post_body, grid=(GRID,),
        in_specs=[_row_spec(din), _row_spec(din), _row_spec(LANE),
                  _full_spec(din, dout), _full_spec(8, dout)],
        out_specs=_row_spec(dout),
        out_shape=jax.ShapeDtypeStruct((N, dout), jnp.float32),
    )(A, g, deg16, W, _b2d(b))


def _tc_post4(A, g, deg16, W4, b4, W5):
    return pl.pallas_call(
        _post4_body, grid=(GRID,),
        in_specs=[_row_spec(256), _row_spec(256), _row_spec(LANE),
                  _full_spec(256, 512), _full_spec(8, 512),
                  _full_spec(512, 128)],
        out_specs=_row_spec(128),
        out_shape=jax.ShapeDtypeStruct((N, 128), jnp.float32),
    )(A, g, deg16, W4, _b2d(b4), W5)


def _tc_final(A, z, deg16, b):
    return pl.pallas_call(
        _final_body, grid=(GRID,),
        in_specs=[_row_spec(128), _row_spec(128), _row_spec(LANE),
                  _full_spec(8, 128)],
        out_specs=_row_spec(128),
        out_shape=jax.ShapeDtypeStruct((N, 128), jnp.float32),
    )(A, z, deg16, _b2d(b))


def kernel(x, edge_index, W1, b1, W2, b2, W3, b3, W4, b4, W5, b5):
    src = edge_index[0].astype(jnp.int32)
    dst = edge_index[1].astype(jnp.int32)
    pad = E_PAD - E
    src2 = jnp.concatenate([src, jnp.zeros((pad,), jnp.int32)]
                           ).reshape(E_PAD // ROW, ROW)
    dst2 = jnp.concatenate([dst, jnp.full((pad,), N, jnp.int32)]
                           ).reshape(E_PAD // ROW, ROW)

    zrnd = jnp.zeros((RND, LANE), jnp.float32)

    def agg(g, nc):
        gflat = g.reshape(N * nc, LANE)
        return _make_agg(nc)(gflat, src2, dst2, zrnd).reshape(N, nc * LANE)

    deg16 = _deg_call(dst2)
    g = _tc_g0(x, deg16)
    g = _tc_post(agg(g, 2), g, deg16, W1, b1)
    g = _tc_post(agg(g, 4), g, deg16, W2, b2)
    g = _tc_post(agg(g, 8), g, deg16, W3, b3)
    z = _tc_post4(agg(g, 16), g, deg16, W4, b4, W5)
    return _tc_final(agg(z, 8), z, deg16, b5)


# speedup vs baseline: 12.6816x; 1.0185x over previous
import functools

import jax
import jax.numpy as jnp
from jax import lax
from jax.experimental import pallas as pl
from jax.experimental.pallas import tpu as pltpu
from jax.experimental.pallas import tpu_sc as plsc

N = 100000
E = 1600000
LANE = 16
N_SC = 2
N_TILES = 16
BLK = 512
ROW = 128
NDMA = BLK // ROW
EPT = -(-E // (N_TILES * 4 * BLK)) * 4 * BLK
E_PAD = EPT * N_TILES
EPT2 = EPT // 2
NBLK2 = EPT2 // BLK
NPT = 6256
NPT_LAST = N - 15 * NPT
RND = 512
NRND = 13
ACC_R = N + 16
NBLK = EPT // BLK
IDXROWS = E_PAD // ROW

_MESH = plsc.VectorSubcoreMesh(core_axis_name="c", subcore_axis_name="s")


def _build_widx(widx, base_node, nc, ch, iota_nc):
    for k in range(RND // ROW):
        for q in range(ROW // LANE):
            off = (base_node + k * ROW + q * LANE) * nc + ch
            widx[k, pl.ds(q * LANE, LANE)] = iota_nc + off


def _round_base(s, m):
    last = jnp.where(s == N_TILES - 1, NPT_LAST - RND, NPT - RND)
    off = jnp.where(m == NRND - 1, last, m * RND)
    return pl.multiple_of(s * NPT + off, 16)


def _agg_body(nc, gflat, src2, dst2, zrnd, oflat, acc,
              sidxA, sidxB, gidxA, gidxB, didx0, didx1, didx2, didx3,
              rowsA, rowsB, widx, bounce, gsem, ssem, isem):
    c = lax.axis_index("c")
    s = lax.axis_index("s")
    nch = nc // N_SC
    iota_nc = lax.iota(jnp.int32, LANE) * nc
    sidx2 = (sidxA, sidxB)
    gidx2 = (gidxA, gidxB)
    didx4 = (didx0, didx1, didx2, didx3)
    rows2 = (rowsA, rowsB)

    def idx_row(jb):
        return jnp.minimum(s * (EPT // ROW) + jb * NDMA, IDXROWS - NDMA)

    def fire_idx(jb, sx, dx):
        r = idx_row(jb)
        pltpu.async_copy(src2.at[pl.ds(r, NDMA)], sx, isem)
        pltpu.async_copy(dst2.at[pl.ds(r, NDMA)], dx, isem)

    def wait_idx(sx, dx):
        pltpu.make_async_copy(src2.at[pl.ds(0, NDMA)], sx, isem).wait()
        pltpu.make_async_copy(dst2.at[pl.ds(0, NDMA)], dx, isem).wait()

    def compute_gidx(gx, sx, ch):
        for k in range(NDMA):
            for q in range(ROW // LANE):
                sl = pl.ds(q * LANE, LANE)
                gx[k, sl] = sx[k, sl] * nc + ch

    def fire_gathers(gx, rx):
        for k in range(NDMA):
            pltpu.async_copy(gflat.at[gx.at[k]],
                             rx.at[pl.ds(k * ROW, ROW)], gsem)

    def wait_gathers(gx, rx):
        for k in range(NDMA):
            pltpu.make_async_copy(gflat.at[gx.at[k]],
                                  rx.at[pl.ds(k * ROW, ROW)], gsem).wait()

    def fire_scatter(rx, dx):
        for k in range(NDMA):
            pltpu.async_copy(rx.at[pl.ds(k * ROW, ROW)],
                             acc.at[dx.at[k]], ssem, add=True)

    def wait_scatter(rx, dx):
        for k in range(NDMA):
            pltpu.make_async_copy(rx.at[pl.ds(k * ROW, ROW)],
                                  acc.at[dx.at[k]], ssem).wait()

    def chunk(j, carry):
        ch = j * N_SC + c
        pltpu.sync_copy(zrnd, bounce)

        def zero_round(m, carry2):
            pltpu.sync_copy(bounce, acc.at[pl.ds(_round_base(s, m), RND)])
            return carry2

        lax.fori_loop(0, NRND, zero_round, 0)
        plsc.subcore_barrier()

        pltpu.sync_copy(src2.at[pl.ds(idx_row(0), NDMA)], sidxA)
        pltpu.sync_copy(dst2.at[pl.ds(idx_row(0), NDMA)], didx0)
        compute_gidx(gidxA, sidxA, ch)
        fire_gathers(gidxA, rowsA)
        fire_idx(1, sidxB, didx1)
        fire_idx(2, sidxA, didx2)

        def quad(i, carry2):
            jb = i * 4
            for t in range(4):
                X, Y = rows2[t % 2], rows2[(t + 1) % 2]
                gX, gY = gidx2[t % 2], gidx2[(t + 1) % 2]
                sY = sidx2[(t + 1) % 2]
                wait_idx(sY, didx4[(t + 1) % 4])
                compute_gidx(gY, sY, ch)
                if t == 0:
                    @pl.when(i > 0)
                    def _():
                        wait_scatter(Y, didx4[3])
                else:
                    wait_scatter(Y, didx4[(t + 3) % 4])
                fire_gathers(gY, Y)
                wait_gathers(gX, X)
                fire_scatter(X, didx4[t % 4])
                fire_idx(jb + t + 3,
                         sidx2[(t + 3) % 2], didx4[(t + 3) % 4])
            return carry2

        lax.fori_loop(0, NBLK // 4, quad, 0)
        wait_gathers(gidx2[NBLK % 2], rows2[NBLK % 2])
        wait_scatter(rows2[(NBLK - 1) % 2], didx4[(NBLK - 1) % 4])
        wait_idx(sidx2[(NBLK + 1) % 2], didx4[(NBLK + 1) % 4])
        wait_idx(sidx2[NBLK % 2], didx4[(NBLK + 2) % 4])
        plsc.subcore_barrier()

        def wout_round(m, carry2):
            b = _round_base(s, m)
            _build_widx(widx, b, nc, ch, iota_nc)
            pltpu.sync_copy(acc.at[pl.ds(b, RND)], bounce)
            cps = [
                pltpu.async_copy(bounce.at[pl.ds(k * ROW, ROW)],
                                 oflat.at[widx.at[k]], gsem)
                for k in range(RND // ROW)
            ]
            for cp in cps:
                cp.wait()
            return carry2

        lax.fori_loop(0, NRND, wout_round, 0)
        plsc.subcore_barrier()
        return carry

    lax.fori_loop(0, nch, chunk, 0)


@functools.cache
def _make_agg(nc):
    idx_t = pltpu.VMEM((NDMA, ROW), jnp.int32)
    rows_t = pltpu.VMEM((BLK, LANE), jnp.float32)
    return pl.kernel(
        functools.partial(_agg_body, nc),
        out_type=jax.ShapeDtypeStruct((N * nc, LANE), jnp.float32),
        mesh=_MESH,
        scratch_types=[
            pltpu.VMEM_SHARED((ACC_R, LANE), jnp.float32),
            idx_t, idx_t, idx_t, idx_t,
            idx_t, idx_t, idx_t, idx_t,
            rows_t, rows_t,
            pltpu.VMEM((RND // ROW, ROW), jnp.int32),
            pltpu.VMEM((RND, LANE), jnp.float32),
            pltpu.SemaphoreType.DMA,
            pltpu.SemaphoreType.DMA,
            pltpu.SemaphoreType.DMA,
        ],
        compiler_params=pltpu.CompilerParams(use_tc_tiling_on_sc=False),
    )


def _deg_body(dst2, zrnd, outa, outb, acc,
              didx0, didx1, didx2, didx3, buf, bounce, ssem, isem):
    c = lax.axis_index("c")
    s = lax.axis_index("s")
    w = c * N_TILES + s
    r0 = s * NPT
    didx4 = (didx0, didx1, didx2, didx3)
    ones = jnp.full((LANE,), 1.0, jnp.float32)
    for k in range(ROW):
        buf[k, :] = ones
    pltpu.sync_copy(zrnd, bounce)

    def zero_round(m, carry):
        pltpu.sync_copy(bounce, acc.at[pl.ds(_round_base(s, m), RND)])
        return carry

    lax.fori_loop(0, NRND, zero_round, 0)
    plsc.subcore_barrier()

    def idx_row(jb):
        return jnp.minimum(w * (EPT2 // ROW) + jb * NDMA, IDXROWS - NDMA)

    def fire_idx(jb, dx):
        pltpu.async_copy(dst2.at[pl.ds(idx_row(jb), NDMA)], dx, isem)

    def wait_idx(dx):
        pltpu.make_async_copy(dst2.at[pl.ds(0, NDMA)], dx, isem).wait()

    def fire_scatter(dx):
        for k in range(NDMA):
            pltpu.async_copy(buf, acc.at[dx.at[k]], ssem, add=True)

    def wait_scatter(dx):
        for k in range(NDMA):
            pltpu.make_async_copy(buf, acc.at[dx.at[k]], ssem).wait()

    fire_idx(0, didx0)
    fire_idx(1, didx1)

    def quad(i, carry):
        jb = i * 4
        for t in range(4):
            wait_idx(didx4[t % 4])
            fire_scatter(didx4[t % 4])
            if t < 2:
                @pl.when(i > 0)
                def _():
                    wait_scatter(didx4[(t + 2) % 4])
            else:
                wait_scatter(didx4[(t + 2) % 4])
            fire_idx(jb + t + 2, didx4[(t + 2) % 4])
        return carry

    lax.fori_loop(0, NBLK2 // 4, quad, 0)
    wait_scatter(didx4[(NBLK2 - 2) % 4])
    wait_scatter(didx4[(NBLK2 - 1) % 4])
    wait_idx(didx4[NBLK2 % 4])
    wait_idx(didx4[(NBLK2 + 1) % 4])
    plsc.subcore_barrier()

    out = (outa, outb)
    for cc in range(N_SC):
        @pl.when(c == cc)
        def _w():
            @pl.when(s < N_TILES - 1)
            def _a():
                pltpu.sync_copy(acc.at[pl.ds(r0, NPT)],
                                out[cc].at[pl.ds(r0, NPT)])
            @pl.when(s == N_TILES - 1)
            def _b():
                pltpu.sync_copy(acc.at[pl.ds(r0, NPT_LAST)],
                                out[cc].at[pl.ds(r0, NPT_LAST)])


_deg_call = pl.kernel(
    _deg_body,
    out_type=(jax.ShapeDtypeStruct((N, LANE), jnp.float32),
              jax.ShapeDtypeStruct((N, LANE), jnp.float32)),
    mesh=_MESH,
    scratch_types=[
        pltpu.VMEM_SHARED((ACC_R, LANE), jnp.float32),
        pltpu.VMEM((NDMA, ROW), jnp.int32),
        pltpu.VMEM((NDMA, ROW), jnp.int32),
        pltpu.VMEM((NDMA, ROW), jnp.int32),
        pltpu.VMEM((NDMA, ROW), jnp.int32),
        pltpu.VMEM((ROW, LANE), jnp.float32),
        pltpu.VMEM((RND, LANE), jnp.float32),
        pltpu.SemaphoreType.DMA,
        pltpu.SemaphoreType.DMA,
    ],
    compiler_params=pltpu.CompilerParams(use_tc_tiling_on_sc=False),
)


BN = 2000
GRID = N // BN
_PREC = lax.Precision.DEFAULT


def _dinv(da_ref, db_ref):
    return lax.rsqrt(da_ref[:, 0:1] + db_ref[:, 0:1] + 1.0)


def _g0_body(x_ref, da_ref, db_ref, o_ref):
    o_ref[...] = x_ref[...] * _dinv(da_ref, db_ref)


def _post_body(A_ref, g_ref, da_ref, db_ref, W_ref, b_ref, o_ref):
    dinv = _dinv(da_ref, db_ref)
    u = (A_ref[...] + g_ref[...]) * dinv
    y = jnp.dot(u, W_ref[...], preferred_element_type=jnp.float32,
                precision=_PREC) + b_ref[0:1, :]
    o_ref[...] = jnp.maximum(y, 0.0) * dinv


def _post4_body(A_ref, g_ref, da_ref, db_ref, W4_ref, b4_ref, W5_ref, o_ref):
    dinv = _dinv(da_ref, db_ref)
    u = (A_ref[...] + g_ref[...]) * dinv
    y = jnp.dot(u, W4_ref[...], preferred_element_type=jnp.float32,
                precision=_PREC) + b4_ref[0:1, :]
    g4 = jnp.maximum(y, 0.0) * dinv
    o_ref[...] = jnp.dot(g4, W5_ref[...], preferred_element_type=jnp.float32,
                         precision=_PREC)


def _final_body(A_ref, g_ref, da_ref, db_ref, b_ref, o_ref):
    o_ref[...] = ((A_ref[...] + g_ref[...]) * _dinv(da_ref, db_ref)
                  + b_ref[0:1, :])


def _row_spec(d):
    return pl.BlockSpec((BN, d), lambda i: (i, 0))


def _full_spec(r, c):
    return pl.BlockSpec((r, c), lambda i: (0, 0))


def _b2d(b):
    return jnp.broadcast_to(b.reshape(1, -1), (8, b.shape[0]))


def _tc_g0(x, dega, degb):
    return pl.pallas_call(
        _g0_body, grid=(GRID,),
        in_specs=[_row_spec(x.shape[1]), _row_spec(LANE), _row_spec(LANE)],
        out_specs=_row_spec(x.shape[1]),
        out_shape=jax.ShapeDtypeStruct((N, x.shape[1]), jnp.float32),
    )(x, dega, degb)


def _tc_post(A, g, dega, degb, W, b):
    din, dout = W.shape
    return pl.pallas_call(
        _post_body, grid=(GRID,),
        in_specs=[_row_spec(din), _row_spec(din), _row_spec(LANE),
                  _row_spec(LANE), _full_spec(din, dout), _full_spec(8, dout)],
        out_specs=_row_spec(dout),
        out_shape=jax.ShapeDtypeStruct((N, dout), jnp.float32),
    )(A, g, dega, degb, W, _b2d(b))


def _tc_post4(A, g, dega, degb, W4, b4, W5):
    return pl.pallas_call(
        _post4_body, grid=(GRID,),
        in_specs=[_row_spec(256), _row_spec(256), _row_spec(LANE),
                  _row_spec(LANE), _full_spec(256, 512), _full_spec(8, 512),
                  _full_spec(512, 128)],
        out_specs=_row_spec(128),
        out_shape=jax.ShapeDtypeStruct((N, 128), jnp.float32),
    )(A, g, dega, degb, W4, _b2d(b4), W5)


def _tc_final(A, z, dega, degb, b):
    return pl.pallas_call(
        _final_body, grid=(GRID,),
        in_specs=[_row_spec(128), _row_spec(128), _row_spec(LANE),
                  _row_spec(LANE), _full_spec(8, 128)],
        out_specs=_row_spec(128),
        out_shape=jax.ShapeDtypeStruct((N, 128), jnp.float32),
    )(A, z, dega, degb, _b2d(b))


def kernel(x, edge_index, W1, b1, W2, b2, W3, b3, W4, b4, W5, b5):
    src = edge_index[0].astype(jnp.int32)
    dst = edge_index[1].astype(jnp.int32)
    pad = E_PAD - E
    src2 = jnp.concatenate([src, jnp.zeros((pad,), jnp.int32)]
                           ).reshape(E_PAD // ROW, ROW)
    dst2 = jnp.concatenate([dst, jnp.full((pad,), N, jnp.int32)]
                           ).reshape(E_PAD // ROW, ROW)

    zrnd = jnp.zeros((RND, LANE), jnp.float32)

    def agg(g, nc):
        gflat = g.reshape(N * nc, LANE)
        return _make_agg(nc)(gflat, src2, dst2, zrnd).reshape(N, nc * LANE)

    da, db = _deg_call(dst2, zrnd)
    g = _tc_g0(x, da, db)
    g = _tc_post(agg(g, 2), g, da, db, W1, b1)
    g = _tc_post(agg(g, 4), g, da, db, W2, b2)
    g = _tc_post(agg(g, 8), g, da, db, W3, b3)
    z = _tc_post4(agg(g, 16), g, da, db, W4, b4, W5)
    return _tc_final(agg(z, 8), z, da, db, b5)
